# Initial kernel scaffold; baseline (speedup 1.0000x reference)
import jax
import jax.numpy as jnp
from jax.experimental import pallas as pl


def kernel(x, L0_W1, L0_b1, L0_g, L0_bt, L0_W2, L0_b2, L1_W1, L1_b1, L1_g, L1_bt, L1_W2, L1_b2, L2_W1, L2_b1, L2_g, L2_bt, L2_W2, L2_b2, lin1_W, lin1_b, lin2_W, lin2_b, edge_index, batch):
    raise NotImplementedError("write your pallas kernel here")


# speedup vs baseline: 4.2023x; 4.2023x over previous
import functools

import jax
import jax.numpy as jnp
from jax import lax
from jax.experimental import pallas as pl
from jax.experimental.pallas import tpu as pltpu
from jax.experimental.pallas import tpu_sc as plsc

N = 10000
E = 320000
D = 128
H = 128
G = 64
OUT = 16

NC = 2
NS = 16
NW = NC * NS
EPT = E // NW
CH = 80
NCHUNK = EPT // CH
N_PAD = 10240
ROWS_PT = N_PAD // NS
ZR = 128

def _sc_segsum_body(h_hbm, src_hbm, dst_hbm, out0, out1,
                    sidx, didx, rows, zbuf, acc, sem):
    cid = lax.axis_index("c")
    sid = lax.axis_index("s")
    wid = sid * NC + cid
    base = wid * EPT

    def zrow(i, carry):
        for c in range(H // 16):
            zbuf[i, pl.ds(c * 16, 16)] = jnp.zeros((16,), jnp.float32)
        return carry
    lax.fori_loop(0, ZR, zrow, 0)
    for j in range(ROWS_PT // ZR):
        pltpu.sync_copy(zbuf, acc.at[pl.ds(sid * ROWS_PT + j * ZR, ZR)])
    plsc.subcore_barrier()

    def chunk(i, carry):
        off = base + i * CH
        pltpu.sync_copy(src_hbm.at[pl.ds(off, CH)], sidx)
        pltpu.sync_copy(dst_hbm.at[pl.ds(off, CH)], didx)
        pltpu.async_copy(h_hbm.at[sidx], rows, sem).wait()
        pltpu.sync_copy(rows, acc.at[didx], add=True)
        return carry
    lax.fori_loop(0, NCHUNK, chunk, 0)
    plsc.subcore_barrier()

    sl = pl.ds(sid * ROWS_PT, ROWS_PT)

    @pl.when(cid == 0)
    def _():
        pltpu.sync_copy(acc.at[sl], out0.at[sl])

    @pl.when(cid == 1)
    def _():
        pltpu.sync_copy(acc.at[sl], out1.at[sl])


@functools.lru_cache(maxsize=None)
def _build_sc_segsum():
    mesh = plsc.VectorSubcoreMesh(core_axis_name="c", subcore_axis_name="s",
                                  num_cores=NC, num_subcores=NS)
    return pl.kernel(
        _sc_segsum_body,
        out_type=(jax.ShapeDtypeStruct((N_PAD, H), jnp.float32),
                  jax.ShapeDtypeStruct((N_PAD, H), jnp.float32)),
        mesh=mesh,
        scratch_types=[
            pltpu.VMEM((CH,), jnp.int32),
            pltpu.VMEM((CH,), jnp.int32),
            pltpu.VMEM((CH, H), jnp.float32),
            pltpu.VMEM((ZR, H), jnp.float32),
            pltpu.VMEM_SHARED((N_PAD, H), jnp.float32),
            pltpu.SemaphoreType.DMA,
        ],
    )


def _sc_segsum(h, src, dst):
    return _build_sc_segsum()(h, src, dst)


def _mlp_body(h_ref, a0_ref, a1_ref, w1_ref, b1_ref, g_ref, bt_ref,
              w2_ref, b2_ref, o_ref):
    hin = h_ref[...] + a0_ref[pl.ds(0, N), :] + a1_ref[pl.ds(0, N), :]
    h1 = jnp.dot(hin, w1_ref[...], preferred_element_type=jnp.float32,
                 precision=lax.Precision.HIGHEST) + b1_ref[...]
    mu = jnp.mean(h1, axis=0, keepdims=True)
    var = jnp.mean(jnp.square(h1 - mu), axis=0, keepdims=True)
    hn = (h1 - mu) * lax.rsqrt(var + 1e-5) * g_ref[...] + bt_ref[...]
    hn = jnp.maximum(hn, 0.0)
    h2 = jnp.dot(hn, w2_ref[...], preferred_element_type=jnp.float32,
                 precision=lax.Precision.HIGHEST) + b2_ref[...]
    o_ref[...] = jnp.maximum(h2, 0.0)


_mlp = pl.pallas_call(
    _mlp_body,
    out_shape=jax.ShapeDtypeStruct((N, H), jnp.float32),
)


def _pool_readout_body(h1_ref, h2_ref, h3_ref, b_ref, w1_ref, b1_ref,
                       w2_ref, b2_ref, o1_ref, o2_ref):
    gids = lax.broadcasted_iota(jnp.int32, (N, G), 1)
    onehot = (b_ref[...] == gids).astype(jnp.float32)
    dn = (((0,), (0,)), ((), ()))
    ones = jnp.ones((N, 1), jnp.float32)
    cnt = lax.dot_general(onehot, ones, dn,
                          preferred_element_type=jnp.float32,
                          precision=lax.Precision.HIGHEST)
    cnt = jnp.maximum(cnt, 1.0)
    s1 = lax.dot_general(onehot, h1_ref[...], dn,
                         preferred_element_type=jnp.float32,
                         precision=lax.Precision.HIGHEST)
    s2 = lax.dot_general(onehot, h2_ref[...], dn,
                         preferred_element_type=jnp.float32,
                         precision=lax.Precision.HIGHEST)
    s3 = lax.dot_general(onehot, h3_ref[...], dn,
                         preferred_element_type=jnp.float32,
                         precision=lax.Precision.HIGHEST)
    hcat = jnp.concatenate([s1, s2, s3], axis=1) / cnt
    hl = jnp.dot(hcat, w1_ref[...], preferred_element_type=jnp.float32,
                 precision=lax.Precision.HIGHEST) + b1_ref[...]
    hl = jnp.maximum(hl, 0.0)
    ho = jnp.dot(hl, w2_ref[...], preferred_element_type=jnp.float32,
                 precision=lax.Precision.HIGHEST) + b2_ref[...]
    o1_ref[...] = ho
    mx = jnp.max(ho, axis=1, keepdims=True)
    lse = jnp.log(jnp.sum(jnp.exp(ho - mx), axis=1, keepdims=True)) + mx
    o2_ref[...] = ho - lse


_pool_readout = pl.pallas_call(
    _pool_readout_body,
    out_shape=(jax.ShapeDtypeStruct((G, OUT), jnp.float32),
               jax.ShapeDtypeStruct((G, OUT), jnp.float32)),
)


def kernel(x, L0_W1, L0_b1, L0_g, L0_bt, L0_W2, L0_b2,
           L1_W1, L1_b1, L1_g, L1_bt, L1_W2, L1_b2,
           L2_W1, L2_b1, L2_g, L2_bt, L2_W2, L2_b2,
           lin1_W, lin1_b, lin2_W, lin2_b, edge_index, batch):
    src = edge_index[0]
    dst = edge_index[1]
    params = [
        (L0_W1, L0_b1, L0_g, L0_bt, L0_W2, L0_b2),
        (L1_W1, L1_b1, L1_g, L1_bt, L1_W2, L1_b2),
        (L2_W1, L2_b1, L2_g, L2_bt, L2_W2, L2_b2),
    ]
    h = x
    hs = []
    for (W1, b1, g, bt, W2, b2) in params:
        a0, a1 = _sc_segsum(h, src, dst)
        h = _mlp(h, a0, a1, W1, b1.reshape(1, H), g.reshape(1, H),
                 bt.reshape(1, H), W2, b2.reshape(1, H))
        hs.append(h)
    return _pool_readout(hs[0], hs[1], hs[2], batch.reshape(N, 1),
                         lin1_W, lin1_b.reshape(1, H * 3),
                         lin2_W, lin2_b.reshape(1, OUT))
